# Initial kernel scaffold; baseline (speedup 1.0000x reference)
import jax
import jax.numpy as jnp
from jax.experimental import pallas as pl


def kernel(x, edge_index, W, b):
    raise NotImplementedError("write your pallas kernel here")


# speedup vs baseline: 127.4268x; 127.4268x over previous
import functools

import jax
import jax.numpy as jnp
from jax import lax
from jax.experimental import pallas as pl
from jax.experimental.pallas import tpu as pltpu
from jax.experimental.pallas import tpu_sc as plsc

_NC = 2
_NS = 16
_NW = _NC * _NS
_L = 16


def _sc_mesh():
    return plsc.VectorSubcoreMesh(
        core_axis_name="c", subcore_axis_name="s",
        num_cores=_NC, num_subcores=_NS)


def _wid():
    return lax.axis_index("s") * _NC + lax.axis_index("c")


def _zero_ref(ref):
    zeros = jnp.zeros((_L,), jnp.float32)

    def body(i, carry):
        ref[pl.ds(i * _L, _L)] = zeros
        return carry

    lax.fori_loop(0, ref.shape[0] // _L, body, 0, unroll=4)


def _deg_body(dst_hbm, out_hbm, dst_v, acc_v):
    epw = dst_v.shape[0]
    wid = _wid()
    pltpu.sync_copy(dst_hbm.at[pl.ds(wid * epw, epw)], dst_v)
    _zero_ref(acc_v)
    ones = jnp.ones((_L,), jnp.float32)

    def body(i, carry):
        d = dst_v[pl.ds(i * _L, _L)]
        plsc.addupdate_scatter(acc_v, [d], ones)
        return carry

    lax.fori_loop(0, epw // _L, body, 0, unroll=4)
    pltpu.sync_copy(acc_v, out_hbm.at[wid])


def _msg_body(src_hbm, dst_hbm, g_hbm, out_hbm, src_v, dst_v, g_v, acc_v):
    epw = src_v.shape[0]
    wid = _wid()
    pltpu.sync_copy(g_hbm, g_v)
    pltpu.sync_copy(src_hbm.at[pl.ds(wid * epw, epw)], src_v)
    pltpu.sync_copy(dst_hbm.at[pl.ds(wid * epw, epw)], dst_v)
    _zero_ref(acc_v)

    def body(i, carry):
        s = src_v[pl.ds(i * _L, _L)]
        d = dst_v[pl.ds(i * _L, _L)]
        vals = plsc.load_gather(g_v, [s])
        plsc.addupdate_scatter(acc_v, [d], vals)
        return carry

    lax.fori_loop(0, epw // _L, body, 0, unroll=4)
    pltpu.sync_copy(acc_v, out_hbm.at[wid])


def _deg_call(dst, n):
    e = dst.shape[0]
    epw = e // _NW
    fn = pl.kernel(
        _deg_body,
        out_type=jax.ShapeDtypeStruct((_NW, n), jnp.float32),
        mesh=_sc_mesh(),
        compiler_params=pltpu.CompilerParams(needs_layout_passes=False),
        scratch_types=[
            pltpu.VMEM((epw,), jnp.int32),
            pltpu.VMEM((n,), jnp.float32),
        ],
    )
    return fn(dst)


def _msg_call(src, dst, g, n):
    e = src.shape[0]
    epw = e // _NW
    fn = pl.kernel(
        _msg_body,
        out_type=jax.ShapeDtypeStruct((_NW, n), jnp.float32),
        mesh=_sc_mesh(),
        compiler_params=pltpu.CompilerParams(needs_layout_passes=False),
        scratch_types=[
            pltpu.VMEM((epw,), jnp.int32),
            pltpu.VMEM((epw,), jnp.int32),
            pltpu.VMEM((n,), jnp.float32),
            pltpu.VMEM((n,), jnp.float32),
        ],
    )
    return fn(src, dst, g)


def _prep_body(x_ref, w_ref, degp_ref, g_ref, dis_ref):
    deg = jnp.sum(degp_ref[...], axis=0, keepdims=True) + 1.0
    dis = lax.rsqrt(deg)
    h = lax.dot_general(w_ref[...], x_ref[...], (((1,), (1,)), ((), ())),
                        preferred_element_type=jnp.float32)
    g_ref[...] = dis * h
    dis_ref[...] = dis


def _prep_call(x, w_row, degp):
    n = x.shape[0]
    shape = jax.ShapeDtypeStruct((1, n), jnp.float32)
    return pl.pallas_call(
        _prep_body,
        out_shape=(shape, shape),
    )(x, w_row, degp)


def _fin_body(accp_ref, g_ref, dis_ref, b_ref, out_ref):
    tot = jnp.sum(accp_ref[...], axis=0, keepdims=True)
    z = dis_ref[...] * (tot + g_ref[...]) + b_ref[0, 0]
    out_ref[...] = jnp.maximum(z, 0.0) + jnp.log1p(jnp.exp(-jnp.abs(z)))


def _fin_call(accp, g_row, dis_row, b):
    n = g_row.shape[1]
    return pl.pallas_call(
        _fin_body,
        out_shape=jax.ShapeDtypeStruct((1, n), jnp.float32),
    )(accp, g_row, dis_row, b.reshape(1, 1))


@jax.jit
def kernel(x, edge_index, W, b):
    n, d = x.shape
    src = edge_index[0]
    dst = edge_index[1]
    degp = _deg_call(dst, n)
    g_row, dis_row = _prep_call(x, W.reshape(1, d), degp)
    accp = _msg_call(src, dst, g_row.reshape(n), n)
    out_row = _fin_call(accp, g_row, dis_row, b)
    return out_row.reshape(n, 1)
